# baseline (speedup 1.0000x reference)
import jax
import jax.numpy as jnp
from jax.experimental import pallas as pl
from jax.experimental.pallas import tpu as pltpu

_TB = 1024
_NN = 2


def _router_block(x_ref, w1_ref, b1_ref, w2_ref, b2_ref, out_ref,
                  xb_ref, acc_ref):
    n = pl.program_id(1)

    @pl.when(n == 0)
    def _():
        xb_ref[...] = x_ref[...].astype(jnp.bfloat16)

    h = jnp.dot(xb_ref[...], w1_ref[...].astype(jnp.bfloat16),
                preferred_element_type=jnp.float32)
    h = jnp.maximum(h + b1_ref[...], 0.0)
    part = jnp.dot(h.astype(jnp.bfloat16), w2_ref[...].astype(jnp.bfloat16),
                   preferred_element_type=jnp.float32)

    @pl.when(n == 0)
    def _():
        acc_ref[...] = part

    @pl.when(n == _NN - 1)
    def _():
        logits = acc_ref[...] + part + b2_ref[...]
        m = jnp.max(logits, axis=-1, keepdims=True)
        e = jnp.exp(logits - m)
        out_ref[...] = e / jnp.sum(e, axis=-1, keepdims=True)


def kernel(hidden_states, W1, b1, W2, b2):
    tokens, hidden = hidden_states.shape
    half = W1.shape[1]
    experts = W2.shape[1]
    cw = half // _NN
    b1r = b1.reshape(1, half)
    b2r = b2.reshape(1, experts)
    return pl.pallas_call(
        _router_block,
        grid=(tokens // _TB, _NN),
        in_specs=[
            pl.BlockSpec((_TB, hidden), lambda i, n: (i, 0)),
            pl.BlockSpec((hidden, cw), lambda i, n: (0, n)),
            pl.BlockSpec((1, cw), lambda i, n: (0, n)),
            pl.BlockSpec((cw, experts), lambda i, n: (n, 0)),
            pl.BlockSpec((1, experts), lambda i, n: (0, 0)),
        ],
        out_specs=pl.BlockSpec((_TB, experts), lambda i, n: (i, 0)),
        out_shape=jax.ShapeDtypeStruct((tokens, experts), jnp.float32),
        scratch_shapes=[
            pltpu.VMEM((_TB, hidden), jnp.bfloat16),
            pltpu.VMEM((_TB, experts), jnp.float32),
        ],
        compiler_params=pltpu.CompilerParams(
            dimension_semantics=("parallel", "arbitrary"),
        ),
    )(hidden_states, W1, b1r, W2, b2r)


# speedup vs baseline: 1.3915x; 1.3915x over previous
import jax
import jax.numpy as jnp
from jax.experimental import pallas as pl
from jax.experimental.pallas import tpu as pltpu

_TB = 2048


def _router_block(x_ref, w1_ref, b1_ref, w2_ref, b2_ref, out_ref, acc_ref):
    i = pl.program_id(0)
    steps = pl.num_programs(0) - 1
    p = jax.lax.rem(i, 2)

    logits = acc_ref[1 - p]
    m = jnp.max(logits, axis=-1, keepdims=True)
    e = jnp.exp(logits - m)
    out_ref[...] = e / jnp.sum(e, axis=-1, keepdims=True)

    @pl.when(i < steps)
    def _():
        x = x_ref[...].astype(jnp.bfloat16)
        h = jnp.dot(x, w1_ref[...].astype(jnp.bfloat16),
                    preferred_element_type=jnp.float32)
        h = jnp.maximum(h + b1_ref[...], 0.0)
        lg = jnp.dot(h.astype(jnp.bfloat16), w2_ref[...].astype(jnp.bfloat16),
                     preferred_element_type=jnp.float32)
        acc_ref[p] = lg + b2_ref[...]


def kernel(hidden_states, W1, b1, W2, b2):
    tokens, hidden = hidden_states.shape
    half = W1.shape[1]
    experts = W2.shape[1]
    b1r = b1.reshape(1, half)
    b2r = b2.reshape(1, experts)
    steps = tokens // _TB
    return pl.pallas_call(
        _router_block,
        grid=(steps + 1,),
        in_specs=[
            pl.BlockSpec((_TB, hidden), lambda i: (jnp.minimum(i, steps - 1), 0)),
            pl.BlockSpec((hidden, half), lambda i: (0, 0)),
            pl.BlockSpec((1, half), lambda i: (0, 0)),
            pl.BlockSpec((half, experts), lambda i: (0, 0)),
            pl.BlockSpec((1, experts), lambda i: (0, 0)),
        ],
        out_specs=pl.BlockSpec((_TB, experts), lambda i: (jnp.maximum(i - 1, 0), 0)),
        out_shape=jax.ShapeDtypeStruct((tokens, experts), jnp.float32),
        scratch_shapes=[
            pltpu.VMEM((2, _TB, experts), jnp.float32),
        ],
        compiler_params=pltpu.CompilerParams(
            dimension_semantics=("arbitrary",),
        ),
    )(hidden_states, W1, b1r, W2, b2r)


# speedup vs baseline: 1.4541x; 1.0449x over previous
import jax
import jax.numpy as jnp
from jax.experimental import pallas as pl
from jax.experimental.pallas import tpu as pltpu

_TB = 2048


def _router_block(x_ref, w1_ref, b1_ref, w2_ref, b2_ref, out_ref):
    x = x_ref[...].astype(jnp.bfloat16)
    h = jnp.dot(x, w1_ref[...].astype(jnp.bfloat16),
                preferred_element_type=jnp.float32)
    h = jnp.maximum(h + b1_ref[...], 0.0)
    logits = jnp.dot(h.astype(jnp.bfloat16), w2_ref[...].astype(jnp.bfloat16),
                     preferred_element_type=jnp.float32)
    logits = logits + b2_ref[...]
    m = jnp.max(logits, axis=-1, keepdims=True)
    e = jnp.exp(logits - m)
    out_ref[...] = e / jnp.sum(e, axis=-1, keepdims=True)


def kernel(hidden_states, W1, b1, W2, b2):
    tokens, hidden = hidden_states.shape
    half = W1.shape[1]
    experts = W2.shape[1]
    b1r = b1.reshape(1, half)
    b2r = b2.reshape(1, experts)
    return pl.pallas_call(
        _router_block,
        grid=(tokens // _TB,),
        in_specs=[
            pl.BlockSpec((_TB, hidden), lambda i: (i, 0)),
            pl.BlockSpec((hidden, half), lambda i: (0, 0)),
            pl.BlockSpec((1, half), lambda i: (0, 0)),
            pl.BlockSpec((half, experts), lambda i: (0, 0)),
            pl.BlockSpec((1, experts), lambda i: (0, 0)),
        ],
        out_specs=pl.BlockSpec((_TB, experts), lambda i: (i, 0)),
        out_shape=jax.ShapeDtypeStruct((tokens, experts), jnp.float32),
        compiler_params=pltpu.CompilerParams(
            dimension_semantics=("parallel",),
        ),
    )(hidden_states, W1, b1r, W2, b2r)


# speedup vs baseline: 1.4574x; 1.0023x over previous
import jax
import jax.numpy as jnp
from jax.experimental import pallas as pl
from jax.experimental.pallas import tpu as pltpu

_TB = 2048


def _router_block(x_ref, w1_ref, b1_ref, w2_ref, b2_ref, out_ref):
    x = x_ref[...].astype(jnp.bfloat16)
    h = jnp.dot(x, w1_ref[...].astype(jnp.bfloat16),
                preferred_element_type=jnp.float32)
    h = jnp.maximum(h + b1_ref[...], 0.0)
    logits = jnp.dot(h.astype(jnp.bfloat16), w2_ref[...].astype(jnp.bfloat16),
                     preferred_element_type=jnp.float32)
    logits = logits + b2_ref[...]
    e = jnp.exp(logits)
    out_ref[...] = e * (1.0 / jnp.sum(e, axis=-1, keepdims=True))


def kernel(hidden_states, W1, b1, W2, b2):
    tokens, hidden = hidden_states.shape
    half = W1.shape[1]
    experts = W2.shape[1]
    b1r = b1.reshape(1, half)
    b2r = b2.reshape(1, experts)
    return pl.pallas_call(
        _router_block,
        grid=(tokens // _TB,),
        in_specs=[
            pl.BlockSpec((_TB, hidden), lambda i: (i, 0)),
            pl.BlockSpec((hidden, half), lambda i: (0, 0)),
            pl.BlockSpec((1, half), lambda i: (0, 0)),
            pl.BlockSpec((half, experts), lambda i: (0, 0)),
            pl.BlockSpec((1, experts), lambda i: (0, 0)),
        ],
        out_specs=pl.BlockSpec((_TB, experts), lambda i: (i, 0)),
        out_shape=jax.ShapeDtypeStruct((tokens, experts), jnp.float32),
        compiler_params=pltpu.CompilerParams(
            dimension_semantics=("parallel",),
        ),
    )(hidden_states, W1, b1r, W2, b2r)
